# baseline (speedup 1.0000x reference)
import functools

import jax
import jax.numpy as jnp
from jax import lax
from jax.experimental import pallas as pl
from jax.experimental.pallas import tpu as pltpu
from jax.experimental.pallas import tpu_sc as plsc

B = 4
K = 20
V = [12001, 3001, 751, 189, 48]
F = [3, 16, 32, 64]
FP = [4, 16, 32, 64]
C = [16, 32, 64, 128]
VA = [12288, 3008, 768, 192]
LATENT = 256

_NW = 32


def _nt(a, b):
    return lax.dot_general(a, b, (((1,), (1,)), ((), ())),
                           preferred_element_type=jnp.float32)


def _nn(a, b):
    return lax.dot_general(a, b, (((1,), (0,)), ((), ())),
                           preferred_element_type=jnp.float32)


def _elu(h):
    return jnp.where(h > 0, h, jnp.exp(h) - 1.0)


def _sc_gather_impl(table, idx, chunk, tc_tiling):
    n = idx.shape[0]
    d = table.shape[1]
    b_per_w = n // _NW
    nchunks = b_per_w // chunk
    mesh = plsc.VectorSubcoreMesh(core_axis_name="c", subcore_axis_name="s")

    @functools.partial(
        pl.kernel,
        out_type=jax.ShapeDtypeStruct((n, d), jnp.float32),
        mesh=mesh,
        compiler_params=pltpu.CompilerParams(use_tc_tiling_on_sc=tc_tiling),
        scratch_types=[
            pltpu.VMEM((b_per_w,), jnp.int32),
            pltpu.VMEM((chunk, d), jnp.float32),
            pltpu.VMEM((chunk, d), jnp.float32),
            pltpu.SemaphoreType.DMA,
            pltpu.SemaphoreType.DMA,
        ],
    )
    def k(table_hbm, idx_hbm, out_hbm, idx_v, rows0, rows1, gsem, wsem):
        wid = lax.axis_index("s") * 2 + lax.axis_index("c")
        base = wid * b_per_w
        pltpu.sync_copy(idx_hbm.at[pl.ds(base, b_per_w)], idx_v)
        bufs = [rows0, rows1]
        wh = [None, None]
        g = pltpu.async_copy(
            table_hbm.at[idx_v.at[pl.ds(0, chunk)]], bufs[0], gsem)
        for c in range(nchunks):
            cur = c % 2
            g.wait()
            if c + 1 < nchunks:
                nxt = (c + 1) % 2
                if wh[nxt] is not None:
                    wh[nxt].wait()
                    wh[nxt] = None
                g = pltpu.async_copy(
                    table_hbm.at[idx_v.at[pl.ds((c + 1) * chunk, chunk)]],
                    bufs[nxt], gsem)
            wh[cur] = pltpu.async_copy(
                bufs[cur], out_hbm.at[pl.ds(base + c * chunk, chunk)], wsem)
        for c in range(2):
            if wh[c] is not None:
                wh[c].wait()

    return k(table, idx)


def _sc_gather(table, idx, chunk):
    return _sc_gather_impl(table, idx, chunk, False)


def _sc_gather_tiled(table, idx, chunk):
    return _sc_gather_impl(table, idx, chunk, True)


def _level0(gr, wcatt, bias2, d0, v_real):
    va, kd = gr.shape
    bc = wcatt.shape[1]
    vo, vi = d0.shape
    blk = 1536
    gsteps = va // blk

    def body(g_ref, w_ref, b_ref, d_ref, o_ref, acc_ref):
        j = pl.program_id(0)
        h = _elu(_nn(g_ref[...], w_ref[...]) + b_ref[...])
        rows = j * blk + lax.broadcasted_iota(jnp.int32, h.shape, 0)
        h = jnp.where(rows < v_real - 1, h, 0.0)

        @pl.when(j == 0)
        def _():
            acc_ref[...] = jnp.zeros_like(acc_ref)

        @pl.when(j < gsteps - 1)
        def _():
            acc_ref[...] += _nn(d_ref[...], h)

        @pl.when(j == gsteps - 1)
        def _():
            d = d_ref[...]
            cols = lax.broadcasted_iota(jnp.int32, d.shape, 1)
            d = jnp.where(cols < vi - j * blk, d, 0.0)
            acc_ref[...] += _nn(d, h)
            o_ref[...] = acc_ref[...]

    return pl.pallas_call(
        body,
        grid=(gsteps,),
        in_specs=[
            pl.BlockSpec((blk, kd), lambda j: (j, 0)),
            pl.BlockSpec((kd, bc), lambda j: (0, 0)),
            pl.BlockSpec((1, bc), lambda j: (0, 0)),
            pl.BlockSpec((vo, blk), lambda j: (0, j)),
        ],
        out_specs=pl.BlockSpec((vo, bc), lambda j: (0, 0)),
        out_shape=jax.ShapeDtypeStruct((vo, bc), jnp.float32),
        scratch_shapes=[pltpu.VMEM((vo, bc), jnp.float32)],
    )(gr, wcatt, bias2, d0)


def _conv0(gr, wcatt, bias2, v_real):
    va, kd = gr.shape
    bc = wcatt.shape[1]
    blk = 1504
    grid = va // blk

    def body(g_ref, w_ref, b_ref, o_ref):
        pid = pl.program_id(0)
        h = _nn(g_ref[...], w_ref[...]) + b_ref[...]
        h = _elu(h)
        rows = pid * blk + lax.broadcasted_iota(jnp.int32, h.shape, 0)
        o_ref[...] = jnp.where(rows < v_real - 1, h, 0.0)

    return pl.pallas_call(
        body,
        grid=(grid,),
        in_specs=[
            pl.BlockSpec((blk, kd), lambda i: (i, 0)),
            pl.BlockSpec((kd, bc), lambda i: (0, 0)),
            pl.BlockSpec((1, bc), lambda i: (0, 0)),
        ],
        out_specs=pl.BlockSpec((blk, bc), lambda i: (i, 0)),
        out_shape=jax.ShapeDtypeStruct((v_real, bc), jnp.float32),
        compiler_params=pltpu.CompilerParams(
            dimension_semantics=("parallel",)),
    )(gr, wcatt, bias2)


def _dmat0(d0, h):
    vo, vi = d0.shape
    bc = h.shape[1]
    blk = 384
    grid = (vo + blk - 1) // blk

    def body(d_ref, h_ref, o_ref):
        o_ref[...] = _nn(d_ref[...], h_ref[...])

    return pl.pallas_call(
        body,
        grid=(grid,),
        in_specs=[
            pl.BlockSpec((blk, vi), lambda i: (i, 0)),
            pl.BlockSpec((vi, bc), lambda i: (0, 0)),
        ],
        out_specs=pl.BlockSpec((blk, bc), lambda i: (i, 0)),
        out_shape=jax.ShapeDtypeStruct((vo, bc), jnp.float32),
        compiler_params=pltpu.CompilerParams(
            dimension_semantics=("parallel",)),
    )(d0, h)


def _conv_down(gr, wcatt, bias2, dmat, v_real):
    va, kd = gr.shape
    bc = wcatt.shape[1]
    vo, vi = dmat.shape
    blk = 752
    gsteps = va // blk

    def body(g_ref, w_ref, b_ref, d_ref, o_ref, h_ref):
        i = pl.program_id(0)
        h = _elu(_nn(g_ref[...], w_ref[...]) + b_ref[...])
        rows = i * blk + lax.broadcasted_iota(jnp.int32, h.shape, 0)
        h_ref[pl.ds(i * blk, blk), :] = jnp.where(rows < v_real - 1, h, 0.0)

        @pl.when(i == gsteps - 1)
        def _():
            hs = lax.slice(h_ref[...], (0, 0), (vi, bc))
            o_ref[...] = _nn(d_ref[...], hs)

    return pl.pallas_call(
        body,
        grid=(gsteps,),
        in_specs=[
            pl.BlockSpec((blk, kd), lambda i: (i, 0)),
            pl.BlockSpec((kd, bc), lambda i: (0, 0)),
            pl.BlockSpec((1, bc), lambda i: (0, 0)),
            pl.BlockSpec((vo, vi), lambda i: (0, 0)),
        ],
        out_specs=pl.BlockSpec((vo, bc), lambda i: (0, 0)),
        out_shape=jax.ShapeDtypeStruct((vo, bc), jnp.float32),
        scratch_shapes=[pltpu.VMEM((va, bc), jnp.float32)],
    )(gr, wcatt, bias2, dmat)


def _conv_down_k(g, wcatt, bias2, dmat, v_real):
    n, d = g.shape
    va = n // K
    bc = wcatt.shape[1]
    vo, vi = dmat.shape

    def body(g_ref, w_ref, b_ref, d_ref, o_ref):
        acc = b_ref[...] * jnp.ones((va, bc), jnp.float32)
        for k in range(K):
            acc = acc + _nn(g_ref[pl.ds(k * va, va), :],
                            w_ref[pl.ds(k * d, d), :])
        h = _elu(acc)
        rows = lax.broadcasted_iota(jnp.int32, h.shape, 0)
        h = jnp.where(rows < v_real - 1, h, 0.0)
        h = lax.slice(h, (0, 0), (vi, bc))
        o_ref[...] = _nn(d_ref[...], h)

    return pl.pallas_call(
        body,
        out_shape=jax.ShapeDtypeStruct((vo, bc), jnp.float32),
    )(g, wcatt, bias2, dmat)


def _level3(table, idx2d, wcatt, bias2, dmat):
    vp = table.shape[0]
    d = table.shape[1]
    bc = wcatt.shape[1]
    vo, vi = dmat.shape

    def body(t_ref, i_ref, w_ref, b_ref, d_ref, o_ref):
        tab = t_ref[...]
        iota = lax.broadcasted_iota(jnp.int32, (vp, vp), 1)
        acc = b_ref[...] * jnp.ones((vp, bc), jnp.float32)
        for k in range(K):
            idxk = i_ref[k * vp:(k + 1) * vp, :]
            pk = jnp.where(idxk == iota, 1.0, 0.0)
            gk = _nn(pk, tab)
            acc = acc + _nn(gk, w_ref[k * d:(k + 1) * d, :])
        h = _elu(acc)
        rows = lax.broadcasted_iota(jnp.int32, h.shape, 0)
        h = jnp.where(rows < vi - 1, h, 0.0)
        h = lax.slice(h, (0, 0), (vi, bc))
        o_ref[...] = _nn(d_ref[...], h)

    return pl.pallas_call(
        body,
        out_shape=jax.ShapeDtypeStruct((vo, bc), jnp.float32),
    )(table, idx2d, wcatt, bias2, dmat)


def _encode(xflat, wenc, benc2):
    def body(x_ref, w_ref, b_ref, o_ref):
        o_ref[...] = _nt(x_ref[...], w_ref[...]) + b_ref[...]

    return pl.pallas_call(
        body,
        out_shape=jax.ShapeDtypeStruct((xflat.shape[0], wenc.shape[0]),
                                       jnp.float32),
    )(xflat, wenc, benc2)


def _build_wcatt(w, i):
    w3 = w.reshape(C[i], K, F[i])
    if FP[i] != F[i]:
        w3 = jnp.pad(w3, ((0, 0), (0, 0), (0, FP[i] - F[i])))
    w4 = w3.transpose(1, 2, 0)
    parts = [jnp.pad(w4, ((0, 0), (0, 0),
                          (b * C[i], (B - 1 - b) * C[i])))[:, None]
             for b in range(B)]
    wcatt = jnp.concatenate(parts, axis=1)
    return wcatt.reshape(K * B * FP[i], B * C[i])


_SC_CHUNK = [1920, 376, 240, 120]


def kernel(x, spiral0, spiral1, spiral2, spiral3,
           W0, b0, D0, W1, b1, D1, W2, b2, D2, W3, b3, D3, Wenc, benc):
    spirals = [spiral0, spiral1, spiral2, spiral3]
    Ws = [W0, W1, W2, W3]
    bs = [b0, b1, b2, b3]
    Ds = [D0, D1, D2, D3]

    xt = jnp.transpose(x, (1, 0, 2))
    xt = jnp.pad(xt, ((0, VA[0] - V[0]), (0, 0), (0, 1)))
    table = xt.reshape(VA[0], B * FP[0])

    for i in range(2):
        idx = jnp.concatenate(
            [spirals[i].reshape(-1),
             jnp.zeros(((VA[i] - V[i]) * K,), jnp.int32)])
        g = _sc_gather(table, idx, _SC_CHUNK[i])
        gr = g.reshape(VA[i], K * B * FP[i])
        wcatt = _build_wcatt(Ws[i], i)
        bias2 = jnp.tile(bs[i], B).reshape(1, B * C[i])
        if i == 0:
            table = _level0(gr, wcatt, bias2, Ds[0], V[0])
        else:
            table = _conv_down(gr, wcatt, bias2, Ds[i], V[i])

    idx2 = jnp.pad(spirals[2], ((0, VA[2] - V[2]), (0, 0)))
    idx2 = idx2.transpose(1, 0).reshape(-1)
    g2 = _sc_gather_tiled(table, idx2, _SC_CHUNK[2])
    bias22 = jnp.tile(bs[2], B).reshape(1, B * C[2])
    table = _conv_down_k(g2, _build_wcatt(Ws[2], 2), bias22, Ds[2], V[2])

    t3 = jnp.pad(table, ((0, VA[3] - V[3]), (0, 0)))
    idx3 = jnp.pad(spirals[3], ((0, VA[3] - V[3]), (0, 0)))
    idx3 = idx3.transpose(1, 0).reshape(K * VA[3], 1)
    bias23 = jnp.tile(bs[3], B).reshape(1, B * C[3])
    table = _level3(t3, idx3, _build_wcatt(Ws[3], 3), bias23, Ds[3])
    xflat = table.reshape(V[4], B, C[3]).transpose(1, 0, 2).reshape(B, -1)
    return _encode(xflat, Wenc, benc.reshape(1, LATENT))


# speedup vs baseline: 1.0137x; 1.0137x over previous
import functools

import jax
import jax.numpy as jnp
from jax import lax
from jax.experimental import pallas as pl
from jax.experimental.pallas import tpu as pltpu
from jax.experimental.pallas import tpu_sc as plsc

B = 4
K = 20
V = [12001, 3001, 751, 189, 48]
F = [3, 16, 32, 64]
FP = [4, 16, 32, 64]
C = [16, 32, 64, 128]
VA = [12288, 3008, 768, 192]
LATENT = 256

_NW = 32


def _nt(a, b):
    return lax.dot_general(a, b, (((1,), (1,)), ((), ())),
                           preferred_element_type=jnp.float32)


def _nn(a, b):
    return lax.dot_general(a, b, (((1,), (0,)), ((), ())),
                           preferred_element_type=jnp.float32)


def _elu(h):
    return jnp.where(h > 0, h, jnp.exp(h) - 1.0)


def _sc_gather_impl(table, idx, chunk, tc_tiling):
    n = idx.shape[0]
    d = table.shape[1]
    b_per_w = n // _NW
    nchunks = b_per_w // chunk
    mesh = plsc.VectorSubcoreMesh(core_axis_name="c", subcore_axis_name="s")

    @functools.partial(
        pl.kernel,
        out_type=jax.ShapeDtypeStruct((n, d), jnp.float32),
        mesh=mesh,
        compiler_params=pltpu.CompilerParams(use_tc_tiling_on_sc=tc_tiling),
        scratch_types=[
            pltpu.VMEM((chunk,), jnp.int32),
            pltpu.VMEM((chunk, d), jnp.float32),
            pltpu.SemaphoreType.DMA,
        ],
    )
    def k(table_hbm, idx_hbm, out_hbm, idx_v, rows_v, sem):
        wid = lax.axis_index("s") * 2 + lax.axis_index("c")
        base = wid * b_per_w

        @pl.loop(0, nchunks)
        def _(ci):
            off = base + ci * chunk
            pltpu.sync_copy(idx_hbm.at[pl.ds(off, chunk)], idx_v)
            pltpu.async_copy(table_hbm.at[idx_v], rows_v, sem).wait()
            pltpu.sync_copy(rows_v, out_hbm.at[pl.ds(off, chunk)])

    return k(table, idx)


def _sc_gather(table, idx, chunk):
    return _sc_gather_impl(table, idx, chunk, False)


def _sc_gather_tiled(table, idx, chunk):
    return _sc_gather_impl(table, idx, chunk, True)


def _level0(gr, wcatt, bias2, d0, v_real):
    va, kd = gr.shape
    bc = wcatt.shape[1]
    vo, vi = d0.shape
    blk = 1536
    gsteps = va // blk

    def body(g_ref, w_ref, b_ref, d_ref, o_ref, acc_ref):
        j = pl.program_id(0)
        h = _elu(_nn(g_ref[...], w_ref[...]) + b_ref[...])
        rows = j * blk + lax.broadcasted_iota(jnp.int32, h.shape, 0)
        h = jnp.where(rows < v_real - 1, h, 0.0)

        @pl.when(j == 0)
        def _():
            acc_ref[...] = jnp.zeros_like(acc_ref)

        @pl.when(j < gsteps - 1)
        def _():
            acc_ref[...] += _nn(d_ref[...], h)

        @pl.when(j == gsteps - 1)
        def _():
            d = d_ref[...]
            cols = lax.broadcasted_iota(jnp.int32, d.shape, 1)
            d = jnp.where(cols < vi - j * blk, d, 0.0)
            acc_ref[...] += _nn(d, h)
            o_ref[...] = acc_ref[...]

    return pl.pallas_call(
        body,
        grid=(gsteps,),
        in_specs=[
            pl.BlockSpec((blk, kd), lambda j: (j, 0)),
            pl.BlockSpec((kd, bc), lambda j: (0, 0)),
            pl.BlockSpec((1, bc), lambda j: (0, 0)),
            pl.BlockSpec((vo, blk), lambda j: (0, j)),
        ],
        out_specs=pl.BlockSpec((vo, bc), lambda j: (0, 0)),
        out_shape=jax.ShapeDtypeStruct((vo, bc), jnp.float32),
        scratch_shapes=[pltpu.VMEM((vo, bc), jnp.float32)],
    )(gr, wcatt, bias2, d0)


def _conv0(gr, wcatt, bias2, v_real):
    va, kd = gr.shape
    bc = wcatt.shape[1]
    blk = 1504
    grid = va // blk

    def body(g_ref, w_ref, b_ref, o_ref):
        pid = pl.program_id(0)
        h = _nn(g_ref[...], w_ref[...]) + b_ref[...]
        h = _elu(h)
        rows = pid * blk + lax.broadcasted_iota(jnp.int32, h.shape, 0)
        o_ref[...] = jnp.where(rows < v_real - 1, h, 0.0)

    return pl.pallas_call(
        body,
        grid=(grid,),
        in_specs=[
            pl.BlockSpec((blk, kd), lambda i: (i, 0)),
            pl.BlockSpec((kd, bc), lambda i: (0, 0)),
            pl.BlockSpec((1, bc), lambda i: (0, 0)),
        ],
        out_specs=pl.BlockSpec((blk, bc), lambda i: (i, 0)),
        out_shape=jax.ShapeDtypeStruct((v_real, bc), jnp.float32),
        compiler_params=pltpu.CompilerParams(
            dimension_semantics=("parallel",)),
    )(gr, wcatt, bias2)


def _dmat0(d0, h):
    vo, vi = d0.shape
    bc = h.shape[1]
    blk = 384
    grid = (vo + blk - 1) // blk

    def body(d_ref, h_ref, o_ref):
        o_ref[...] = _nn(d_ref[...], h_ref[...])

    return pl.pallas_call(
        body,
        grid=(grid,),
        in_specs=[
            pl.BlockSpec((blk, vi), lambda i: (i, 0)),
            pl.BlockSpec((vi, bc), lambda i: (0, 0)),
        ],
        out_specs=pl.BlockSpec((blk, bc), lambda i: (i, 0)),
        out_shape=jax.ShapeDtypeStruct((vo, bc), jnp.float32),
        compiler_params=pltpu.CompilerParams(
            dimension_semantics=("parallel",)),
    )(d0, h)


def _conv_down(gr, wcatt, bias2, dmat, v_real):
    va, kd = gr.shape
    bc = wcatt.shape[1]
    vo, vi = dmat.shape
    blk = 752
    gsteps = va // blk

    def body(g_ref, w_ref, b_ref, d_ref, o_ref, h_ref):
        i = pl.program_id(0)
        h = _elu(_nn(g_ref[...], w_ref[...]) + b_ref[...])
        rows = i * blk + lax.broadcasted_iota(jnp.int32, h.shape, 0)
        h_ref[pl.ds(i * blk, blk), :] = jnp.where(rows < v_real - 1, h, 0.0)

        @pl.when(i == gsteps - 1)
        def _():
            hs = lax.slice(h_ref[...], (0, 0), (vi, bc))
            o_ref[...] = _nn(d_ref[...], hs)

    return pl.pallas_call(
        body,
        grid=(gsteps,),
        in_specs=[
            pl.BlockSpec((blk, kd), lambda i: (i, 0)),
            pl.BlockSpec((kd, bc), lambda i: (0, 0)),
            pl.BlockSpec((1, bc), lambda i: (0, 0)),
            pl.BlockSpec((vo, vi), lambda i: (0, 0)),
        ],
        out_specs=pl.BlockSpec((vo, bc), lambda i: (0, 0)),
        out_shape=jax.ShapeDtypeStruct((vo, bc), jnp.float32),
        scratch_shapes=[pltpu.VMEM((va, bc), jnp.float32)],
    )(gr, wcatt, bias2, dmat)


def _conv_down_k(g, wcatt, bias2, dmat, v_real):
    n, d = g.shape
    va = n // K
    bc = wcatt.shape[1]
    vo, vi = dmat.shape

    def body(g_ref, w_ref, b_ref, d_ref, o_ref):
        acc = b_ref[...] * jnp.ones((va, bc), jnp.float32)
        for k in range(K):
            acc = acc + _nn(g_ref[pl.ds(k * va, va), :],
                            w_ref[pl.ds(k * d, d), :])
        h = _elu(acc)
        rows = lax.broadcasted_iota(jnp.int32, h.shape, 0)
        h = jnp.where(rows < v_real - 1, h, 0.0)
        h = lax.slice(h, (0, 0), (vi, bc))
        o_ref[...] = _nn(d_ref[...], h)

    return pl.pallas_call(
        body,
        out_shape=jax.ShapeDtypeStruct((vo, bc), jnp.float32),
    )(g, wcatt, bias2, dmat)


def _level3(table, idx2d, wcatt, bias2, dmat):
    vp = table.shape[0]
    d = table.shape[1]
    bc = wcatt.shape[1]
    vo, vi = dmat.shape

    def body(t_ref, i_ref, w_ref, b_ref, d_ref, o_ref):
        tab = t_ref[...]
        iota = lax.broadcasted_iota(jnp.int32, (vp, vp), 1)
        acc = b_ref[...] * jnp.ones((vp, bc), jnp.float32)
        for k in range(K):
            idxk = i_ref[k * vp:(k + 1) * vp, :]
            pk = jnp.where(idxk == iota, 1.0, 0.0)
            gk = _nn(pk, tab)
            acc = acc + _nn(gk, w_ref[k * d:(k + 1) * d, :])
        h = _elu(acc)
        rows = lax.broadcasted_iota(jnp.int32, h.shape, 0)
        h = jnp.where(rows < vi - 1, h, 0.0)
        h = lax.slice(h, (0, 0), (vi, bc))
        o_ref[...] = _nn(d_ref[...], h)

    return pl.pallas_call(
        body,
        out_shape=jax.ShapeDtypeStruct((vo, bc), jnp.float32),
    )(table, idx2d, wcatt, bias2, dmat)


def _encode(xflat, wenc, benc2):
    def body(x_ref, w_ref, b_ref, o_ref):
        o_ref[...] = _nt(x_ref[...], w_ref[...]) + b_ref[...]

    return pl.pallas_call(
        body,
        out_shape=jax.ShapeDtypeStruct((xflat.shape[0], wenc.shape[0]),
                                       jnp.float32),
    )(xflat, wenc, benc2)


def _build_wcatt(w, i):
    w3 = w.reshape(C[i], K, F[i])
    if FP[i] != F[i]:
        w3 = jnp.pad(w3, ((0, 0), (0, 0), (0, FP[i] - F[i])))
    w4 = w3.transpose(1, 2, 0)
    parts = [jnp.pad(w4, ((0, 0), (0, 0),
                          (b * C[i], (B - 1 - b) * C[i])))[:, None]
             for b in range(B)]
    wcatt = jnp.concatenate(parts, axis=1)
    return wcatt.reshape(K * B * FP[i], B * C[i])


_SC_CHUNK = [1920, 1880, 480, 120]


def kernel(x, spiral0, spiral1, spiral2, spiral3,
           W0, b0, D0, W1, b1, D1, W2, b2, D2, W3, b3, D3, Wenc, benc):
    spirals = [spiral0, spiral1, spiral2, spiral3]
    Ws = [W0, W1, W2, W3]
    bs = [b0, b1, b2, b3]
    Ds = [D0, D1, D2, D3]

    xt = jnp.transpose(x, (1, 0, 2))
    xt = jnp.pad(xt, ((0, VA[0] - V[0]), (0, 0), (0, 1)))
    table = xt.reshape(VA[0], B * FP[0])

    for i in range(2):
        idx = jnp.concatenate(
            [spirals[i].reshape(-1),
             jnp.zeros(((VA[i] - V[i]) * K,), jnp.int32)])
        g = _sc_gather(table, idx, _SC_CHUNK[i])
        gr = g.reshape(VA[i], K * B * FP[i])
        wcatt = _build_wcatt(Ws[i], i)
        bias2 = jnp.tile(bs[i], B).reshape(1, B * C[i])
        if i == 0:
            table = _level0(gr, wcatt, bias2, Ds[0], V[0])
        else:
            table = _conv_down(gr, wcatt, bias2, Ds[i], V[i])

    idx2 = jnp.pad(spirals[2], ((0, VA[2] - V[2]), (0, 0)))
    idx2 = idx2.transpose(1, 0).reshape(-1)
    g2 = _sc_gather_tiled(table, idx2, _SC_CHUNK[2])
    bias22 = jnp.tile(bs[2], B).reshape(1, B * C[2])
    table = _conv_down_k(g2, _build_wcatt(Ws[2], 2), bias22, Ds[2], V[2])

    t3 = jnp.pad(table, ((0, VA[3] - V[3]), (0, 0)))
    idx3 = jnp.pad(spirals[3], ((0, VA[3] - V[3]), (0, 0)))
    idx3 = idx3.transpose(1, 0).reshape(K * VA[3], 1)
    bias23 = jnp.tile(bs[3], B).reshape(1, B * C[3])
    table = _level3(t3, idx3, _build_wcatt(Ws[3], 3), bias23, Ds[3])
    xflat = table.reshape(V[4], B, C[3]).transpose(1, 0, 2).reshape(B, -1)
    return _encode(xflat, Wenc, benc.reshape(1, LATENT))


# speedup vs baseline: 1.1036x; 1.0887x over previous
import functools

import jax
import jax.numpy as jnp
from jax import lax
from jax.experimental import pallas as pl
from jax.experimental.pallas import tpu as pltpu
from jax.experimental.pallas import tpu_sc as plsc

B = 4
K = 20
V = [12001, 3001, 751, 189, 48]
F = [3, 16, 32, 64]
FP = [4, 16, 32, 64]
C = [16, 32, 64, 128]
VA = [12032, 3008, 768, 192]
LATENT = 256

_NW = 32


def _nt(a, b):
    return lax.dot_general(a, b, (((1,), (1,)), ((), ())),
                           preferred_element_type=jnp.float32)


def _nn(a, b):
    return lax.dot_general(a, b, (((1,), (0,)), ((), ())),
                           preferred_element_type=jnp.float32)


def _elu(h):
    return jnp.where(h > 0, h, jnp.exp(h) - 1.0)


def _sc_gather_impl(table, idx, chunk, tc_tiling):
    n = idx.shape[0]
    d = table.shape[1]
    b_per_w = n // _NW
    nchunks = b_per_w // chunk
    mesh = plsc.VectorSubcoreMesh(core_axis_name="c", subcore_axis_name="s")

    @functools.partial(
        pl.kernel,
        out_type=jax.ShapeDtypeStruct((n, d), jnp.float32),
        mesh=mesh,
        compiler_params=pltpu.CompilerParams(use_tc_tiling_on_sc=tc_tiling),
        scratch_types=[
            pltpu.VMEM((chunk,), jnp.int32),
            pltpu.VMEM((chunk, d), jnp.float32),
            pltpu.SemaphoreType.DMA,
        ],
    )
    def k(table_hbm, idx_hbm, out_hbm, idx_v, rows_v, sem):
        wid = lax.axis_index("s") * 2 + lax.axis_index("c")
        base = wid * b_per_w

        @pl.loop(0, nchunks)
        def _(ci):
            off = base + ci * chunk
            pltpu.sync_copy(idx_hbm.at[pl.ds(off, chunk)], idx_v)
            pltpu.async_copy(table_hbm.at[idx_v], rows_v, sem).wait()
            pltpu.sync_copy(rows_v, out_hbm.at[pl.ds(off, chunk)])

    return k(table, idx)


def _sc_gather(table, idx, chunk):
    return _sc_gather_impl(table, idx, chunk, False)


def _sc_gather_tiled(table, idx, chunk):
    return _sc_gather_impl(table, idx, chunk, True)


def _conv0(gr, wcatt, bias2, v_real):
    va, kd = gr.shape
    bc = wcatt.shape[1]
    blk = 1504
    grid = va // blk

    def body(g_ref, w_ref, b_ref, o_ref):
        pid = pl.program_id(0)
        h = _nn(g_ref[...], w_ref[...]) + b_ref[...]
        h = _elu(h)
        rows = pid * blk + lax.broadcasted_iota(jnp.int32, h.shape, 0)
        o_ref[...] = jnp.where(rows < v_real - 1, h, 0.0)

    return pl.pallas_call(
        body,
        grid=(grid,),
        in_specs=[
            pl.BlockSpec((blk, kd), lambda i: (i, 0)),
            pl.BlockSpec((kd, bc), lambda i: (0, 0)),
            pl.BlockSpec((1, bc), lambda i: (0, 0)),
        ],
        out_specs=pl.BlockSpec((blk, bc), lambda i: (i, 0)),
        out_shape=jax.ShapeDtypeStruct((v_real, bc), jnp.float32),
        compiler_params=pltpu.CompilerParams(
            dimension_semantics=("parallel",)),
    )(gr, wcatt, bias2)


def _dmat0(d0, h):
    vo, vi = d0.shape
    bc = h.shape[1]
    blk = 384
    grid = (vo + blk - 1) // blk

    def body(d_ref, h_ref, o_ref):
        o_ref[...] = _nn(d_ref[...], h_ref[...])

    return pl.pallas_call(
        body,
        grid=(grid,),
        in_specs=[
            pl.BlockSpec((blk, vi), lambda i: (i, 0)),
            pl.BlockSpec((vi, bc), lambda i: (0, 0)),
        ],
        out_specs=pl.BlockSpec((blk, bc), lambda i: (i, 0)),
        out_shape=jax.ShapeDtypeStruct((vo, bc), jnp.float32),
        compiler_params=pltpu.CompilerParams(
            dimension_semantics=("parallel",)),
    )(d0, h)


def _conv_down(gr, wcatt, bias2, dmat, v_real):
    va, kd = gr.shape
    bc = wcatt.shape[1]
    vo, vi = dmat.shape
    blk = 752
    gsteps = va // blk

    def body(g_ref, w_ref, b_ref, d_ref, o_ref, h_ref):
        i = pl.program_id(0)
        h = _elu(_nn(g_ref[...], w_ref[...]) + b_ref[...])
        rows = i * blk + lax.broadcasted_iota(jnp.int32, h.shape, 0)
        h_ref[pl.ds(i * blk, blk), :] = jnp.where(rows < v_real - 1, h, 0.0)

        @pl.when(i == gsteps - 1)
        def _():
            hs = lax.slice(h_ref[...], (0, 0), (vi, bc))
            o_ref[...] = _nn(d_ref[...], hs)

    return pl.pallas_call(
        body,
        grid=(gsteps,),
        in_specs=[
            pl.BlockSpec((blk, kd), lambda i: (i, 0)),
            pl.BlockSpec((kd, bc), lambda i: (0, 0)),
            pl.BlockSpec((1, bc), lambda i: (0, 0)),
            pl.BlockSpec((vo, vi), lambda i: (0, 0)),
        ],
        out_specs=pl.BlockSpec((vo, bc), lambda i: (0, 0)),
        out_shape=jax.ShapeDtypeStruct((vo, bc), jnp.float32),
        scratch_shapes=[pltpu.VMEM((va, bc), jnp.float32)],
    )(gr, wcatt, bias2, dmat)


def _conv_down_k(g, wcatt, bias2, dmat, v_real):
    n, d = g.shape
    va = n // K
    bc = wcatt.shape[1]
    vo, vi = dmat.shape

    def body(g_ref, w_ref, b_ref, d_ref, o_ref):
        acc = b_ref[...] * jnp.ones((va, bc), jnp.float32)
        for k in range(K):
            acc = acc + _nn(g_ref[pl.ds(k * va, va), :],
                            w_ref[pl.ds(k * d, d), :])
        h = _elu(acc)
        rows = lax.broadcasted_iota(jnp.int32, h.shape, 0)
        h = jnp.where(rows < v_real - 1, h, 0.0)
        h = lax.slice(h, (0, 0), (vi, bc))
        o_ref[...] = _nn(d_ref[...], h)

    return pl.pallas_call(
        body,
        out_shape=jax.ShapeDtypeStruct((vo, bc), jnp.float32),
    )(g, wcatt, bias2, dmat)


def _level3(table, idx2d, wcatt, bias2, dmat):
    vp = table.shape[0]
    d = table.shape[1]
    bc = wcatt.shape[1]
    vo, vi = dmat.shape

    def body(t_ref, i_ref, w_ref, b_ref, d_ref, o_ref):
        tab = t_ref[...]
        iota = lax.broadcasted_iota(jnp.int32, (vp, vp), 1)
        acc = b_ref[...] * jnp.ones((vp, bc), jnp.float32)
        for k in range(K):
            idxk = i_ref[k * vp:(k + 1) * vp, :]
            pk = jnp.where(idxk == iota, 1.0, 0.0)
            gk = _nn(pk, tab)
            acc = acc + _nn(gk, w_ref[k * d:(k + 1) * d, :])
        h = _elu(acc)
        rows = lax.broadcasted_iota(jnp.int32, h.shape, 0)
        h = jnp.where(rows < vi - 1, h, 0.0)
        h = lax.slice(h, (0, 0), (vi, bc))
        o_ref[...] = _nn(d_ref[...], h)

    return pl.pallas_call(
        body,
        out_shape=jax.ShapeDtypeStruct((vo, bc), jnp.float32),
    )(table, idx2d, wcatt, bias2, dmat)


def _encode(xflat, wenc, benc2):
    def body(x_ref, w_ref, b_ref, o_ref):
        o_ref[...] = _nt(x_ref[...], w_ref[...]) + b_ref[...]

    return pl.pallas_call(
        body,
        out_shape=jax.ShapeDtypeStruct((xflat.shape[0], wenc.shape[0]),
                                       jnp.float32),
    )(xflat, wenc, benc2)


def _build_wcatt(w, i):
    w3 = w.reshape(C[i], K, F[i])
    if FP[i] != F[i]:
        w3 = jnp.pad(w3, ((0, 0), (0, 0), (0, FP[i] - F[i])))
    w4 = w3.transpose(1, 2, 0)
    parts = [jnp.pad(w4, ((0, 0), (0, 0),
                          (b * C[i], (B - 1 - b) * C[i])))[:, None]
             for b in range(B)]
    wcatt = jnp.concatenate(parts, axis=1)
    return wcatt.reshape(K * B * FP[i], B * C[i])


_SC_CHUNK = [1880, 1880, 480, 120]


def kernel(x, spiral0, spiral1, spiral2, spiral3,
           W0, b0, D0, W1, b1, D1, W2, b2, D2, W3, b3, D3, Wenc, benc):
    spirals = [spiral0, spiral1, spiral2, spiral3]
    Ws = [W0, W1, W2, W3]
    bs = [b0, b1, b2, b3]
    Ds = [D0, D1, D2, D3]

    xt = jnp.transpose(x, (1, 0, 2))
    xt = jnp.pad(xt, ((0, VA[0] - V[0]), (0, 0), (0, 1)))
    table = xt.reshape(VA[0], B * FP[0])

    for i in range(2):
        idx = jnp.concatenate(
            [spirals[i].reshape(-1),
             jnp.zeros(((VA[i] - V[i]) * K,), jnp.int32)])
        g = _sc_gather(table, idx, _SC_CHUNK[i])
        gr = g.reshape(VA[i], K * B * FP[i])
        wcatt = _build_wcatt(Ws[i], i)
        bias2 = jnp.tile(bs[i], B).reshape(1, B * C[i])
        if i == 0:
            h = _conv0(gr, wcatt, bias2, V[0])
            table = _dmat0(Ds[0], h)
        else:
            table = _conv_down(gr, wcatt, bias2, Ds[i], V[i])

    idx2 = jnp.pad(spirals[2], ((0, VA[2] - V[2]), (0, 0)))
    idx2 = idx2.transpose(1, 0).reshape(-1)
    g2 = _sc_gather_tiled(table, idx2, _SC_CHUNK[2])
    bias22 = jnp.tile(bs[2], B).reshape(1, B * C[2])
    table = _conv_down_k(g2, _build_wcatt(Ws[2], 2), bias22, Ds[2], V[2])

    t3 = jnp.pad(table, ((0, VA[3] - V[3]), (0, 0)))
    idx3 = jnp.pad(spirals[3], ((0, VA[3] - V[3]), (0, 0)))
    idx3 = idx3.transpose(1, 0).reshape(K * VA[3], 1)
    bias23 = jnp.tile(bs[3], B).reshape(1, B * C[3])
    table = _level3(t3, idx3, _build_wcatt(Ws[3], 3), bias23, Ds[3])
    xflat = table.reshape(V[4], B, C[3]).transpose(1, 0, 2).reshape(B, -1)
    return _encode(xflat, Wenc, benc.reshape(1, LATENT))
